# baseline (speedup 1.0000x reference)
import jax
import jax.numpy as jnp
from jax.experimental import pallas as pl


def _hw_kernel(h_ref, w_ref, o_ref):
    o_ref[...] = jnp.dot(h_ref[...], w_ref[...],
                         preferred_element_type=jnp.float32)


def _ahw_kernel(a_ref, hw_ref, b_ref, o_ref):
    o_ref[...] = jnp.dot(a_ref[...], hw_ref[...],
                         preferred_element_type=jnp.float32) + b_ref[...]


def kernel(structure, H, W, bias):
    n, a_dim = H.shape
    b_dim = W.shape[1]

    hw = pl.pallas_call(
        _hw_kernel,
        out_shape=jax.ShapeDtypeStruct((n, b_dim), jnp.float32),
    )(H, W)

    bm = 400
    out = pl.pallas_call(
        _ahw_kernel,
        grid=(n // bm,),
        in_specs=[
            pl.BlockSpec((bm, n), lambda i: (i, 0)),
            pl.BlockSpec((n, b_dim), lambda i: (0, 0)),
            pl.BlockSpec((1, b_dim), lambda i: (0, 0)),
        ],
        out_specs=pl.BlockSpec((bm, b_dim), lambda i: (i, 0)),
        out_shape=jax.ShapeDtypeStruct((n, b_dim), jnp.float32),
    )(structure, hw, bias.reshape(1, b_dim))
    return out


# speedup vs baseline: 1.0470x; 1.0470x over previous
import jax
import jax.numpy as jnp
from jax.experimental import pallas as pl
from jax.experimental.pallas import tpu as pltpu


def _fused_kernel(a_ref, h_ref, w_ref, b_ref, o_ref, hw_ref):
    @pl.when(pl.program_id(0) == 0)
    def _():
        hw_ref[...] = jnp.dot(h_ref[...], w_ref[...],
                              preferred_element_type=jnp.float32)

    o_ref[...] = jnp.dot(a_ref[...], hw_ref[...],
                         preferred_element_type=jnp.float32) + b_ref[...]


def kernel(structure, H, W, bias):
    n, a_dim = H.shape
    b_dim = W.shape[1]

    bm = 400
    out = pl.pallas_call(
        _fused_kernel,
        grid=(n // bm,),
        in_specs=[
            pl.BlockSpec((bm, n), lambda i: (i, 0)),
            pl.BlockSpec((n, a_dim), lambda i: (0, 0)),
            pl.BlockSpec((a_dim, b_dim), lambda i: (0, 0)),
            pl.BlockSpec((1, b_dim), lambda i: (0, 0)),
        ],
        out_specs=pl.BlockSpec((bm, b_dim), lambda i: (i, 0)),
        out_shape=jax.ShapeDtypeStruct((n, b_dim), jnp.float32),
        scratch_shapes=[pltpu.VMEM((n, b_dim), jnp.float32)],
    )(structure, H, W, bias.reshape(1, b_dim))
    return out
